# Initial kernel scaffold; baseline (speedup 1.0000x reference)
import jax
import jax.numpy as jnp
from jax.experimental import pallas as pl


def kernel(x, edge_index, Wq, Wk, Wv, Wo, gamma_attn, beta_attn, gamma1, beta1, W1, b1, W2, b2, gamma2, beta2, Wp, bp):
    raise NotImplementedError("write your pallas kernel here")


# speedup vs baseline: 18.5247x; 18.5247x over previous
import functools

import jax
import jax.numpy as jnp
from jax import lax
from jax.experimental import pallas as pl
from jax.experimental.pallas import tpu as pltpu
from jax.experimental.pallas import tpu_sc as plsc

_NC = 2
_NS = 16
_NW = _NC * _NS
_CH = 80
_GRP = 16


def _bn(y, g, b):
    m = jnp.mean(y, axis=0)
    v = jnp.mean((y - m) ** 2, axis=0)
    return (y - m) / jnp.sqrt(v + 1e-5) * g + b


def _tc_qkv(x, wq, wk, wv, scale):
    n, d = x.shape

    def body(x_ref, wq_ref, wk_ref, wv_ref, q_ref, k_ref, v_ref):
        xv = x_ref[...]
        q_ref[...] = jnp.dot(xv, wq_ref[...],
                             preferred_element_type=jnp.float32) * scale
        k_ref[...] = jnp.dot(xv, wk_ref[...],
                             preferred_element_type=jnp.float32)
        v_ref[...] = jnp.dot(xv, wv_ref[...],
                             preferred_element_type=jnp.float32)

    return pl.pallas_call(
        body,
        out_shape=(jax.ShapeDtypeStruct((n, d), jnp.float32),
                   jax.ShapeDtypeStruct((n, d), jnp.float32),
                   jax.ShapeDtypeStruct((n, d), jnp.float32)),
    )(x, wq, wk, wv)


def _sc_edge_pass(q, k, v, src, dst, zeros, n, d, h, accw):
    e = src.shape[0]
    epw = e // _NW
    nchunk = epw // _CH
    ngrp = _CH // _GRP
    npad = ((n + 127) // 128) * 128
    npc = npad // _NS
    dh = d // h
    mesh = plsc.VectorSubcoreMesh(core_axis_name="c", subcore_axis_name="s")

    @functools.partial(
        pl.kernel,
        out_type=jax.ShapeDtypeStruct((_NC, npad, accw), jnp.float32),
        mesh=mesh,
        compiler_params=pltpu.CompilerParams(
            use_tc_tiling_on_sc=False, needs_layout_passes=False),
        scratch_types=[
            pltpu.VMEM_SHARED((npad, accw), jnp.float32),
            pltpu.VMEM((_CH,), jnp.int32),
            pltpu.VMEM((_CH,), jnp.int32),
            pltpu.VMEM((_CH, d), jnp.float32),
            pltpu.VMEM((_CH, d), jnp.float32),
            pltpu.VMEM((_CH, accw), jnp.float32),
            pltpu.VMEM((16, _CH), jnp.float32),
            pltpu.SemaphoreType.DMA,
            pltpu.SemaphoreType.DMA,
        ],
    )
    def body(q_ref, k_ref, v_ref, src_ref, dst_ref, zero_ref, out_ref,
             acc, srcv, dstv, qr, sbuf, msg, escr, sem1, sem2):
        cid = lax.axis_index("c")
        sid = lax.axis_index("s")
        wid = cid * _NS + sid
        pltpu.sync_copy(zero_ref, acc.at[pl.ds(sid * npc, npc)])
        for r in range(16):
            for cc in range(_CH // 16):
                escr[r, pl.ds(cc * 16, 16)] = jnp.zeros((16,), jnp.float32)
        plsc.subcore_barrier()

        base0 = wid * epw
        iota = lax.iota(jnp.int32, 16)

        @pl.loop(0, nchunk)
        def _chunk(i):
            base = base0 + i * _CH
            pltpu.sync_copy(src_ref.at[pl.ds(base, _CH)], srcv)
            pltpu.sync_copy(dst_ref.at[pl.ds(base, _CH)], dstv)
            cp1 = pltpu.async_copy(q_ref.at[dstv], qr, sem1)
            cp2 = pltpu.async_copy(k_ref.at[srcv], sbuf, sem2)
            cp1.wait()
            cp2.wait()

            @pl.loop(0, ngrp)
            def _group(g):
                rows = g * _GRP + iota
                for hh in range(h):
                    s = jnp.zeros((16,), jnp.float32)
                    for j in range(dh):
                        col = jnp.full((16,), hh * dh + j, jnp.int32)
                        a = plsc.load_gather(qr, [rows, col])
                        b = plsc.load_gather(sbuf, [rows, col])
                        s = s + a * b
                    plsc.store_scatter(escr,
                                       [jnp.full((16,), hh, jnp.int32), rows],
                                       jnp.exp(s))

            pltpu.async_copy(v_ref.at[srcv], sbuf, sem2).wait()

            @pl.loop(0, ngrp)
            def _group2(g):
                for ee in range(_GRP):
                    row = g * _GRP + ee
                    rowv = jnp.full((16,), row, jnp.int32)
                    es = plsc.load_gather(escr, [iota, rowv])
                    plsc.store_scatter(msg, [rowv, d + iota], es)
                    for hh in range(h):
                        hsel = jnp.full((16,), hh, jnp.int32)
                        sph = plsc.load_gather(escr, [hsel, rowv])
                        vh = plsc.load_gather(sbuf, [rowv, hh * dh + iota])
                        plsc.store_scatter(msg, [rowv, hh * dh + iota],
                                           sph * vh)

            pltpu.sync_copy(msg, acc.at[dstv], add=True)

        plsc.subcore_barrier()
        pltpu.sync_copy(acc.at[pl.ds(sid * npc, npc)],
                        out_ref.at[cid, pl.ds(sid * npc, npc)])

    return body(q, k, v, src, dst, zeros)


def _tc_dense(x, acc2, wo, ga, ba, g1, be1, w1, bb1, w2, bb2, g2, be2):
    n, d = x.shape
    h = 8
    dh = d // h

    def body(x_ref, acc_ref, wo_ref, ga_ref, ba_ref, g1_ref, be1_ref,
             w1_ref, bb1_ref, w2_ref, bb2_ref, g2_ref, be2_ref, out_ref):
        xv = x_ref[...]
        u = acc_ref[0, :n, :d] + acc_ref[1, :n, :d]
        den = acc_ref[0, :n, d:d + h] + acc_ref[1, :n, d:d + h]
        ind = (lax.broadcasted_iota(jnp.int32, (h, d), 1) // dh
               == lax.broadcasted_iota(jnp.int32, (h, d), 0)
               ).astype(jnp.float32)
        denf = jnp.dot(den, ind, preferred_element_type=jnp.float32)
        agg = u / (denf + 1e-16)
        y0 = jnp.dot(agg, wo_ref[...], preferred_element_type=jnp.float32) + xv
        a1 = _bn(y0, ga_ref[...], ba_ref[...])
        x1 = _bn(a1 + xv, g1_ref[...], be1_ref[...])
        hid = jnp.maximum(
            jnp.dot(x1, w1_ref[...], preferred_element_type=jnp.float32)
            + bb1_ref[...], 0.0)
        h2 = jnp.dot(hid, w2_ref[...],
                     preferred_element_type=jnp.float32) + bb2_ref[...]
        out_ref[...] = _bn(h2 + x1, g2_ref[...], be2_ref[...])

    return pl.pallas_call(
        body,
        out_shape=jax.ShapeDtypeStruct((n, d), jnp.float32),
    )(x, acc2, wo, ga, ba, g1, be1, w1, bb1, w2, bb2, g2, be2)


def _tc_final(x, wp, bp):
    n, d = x.shape
    c = wp.shape[1]

    def body(x_ref, wp_ref, bp_ref, out_ref):
        out_ref[...] = (jnp.dot(x_ref[...], wp_ref[...],
                                preferred_element_type=jnp.float32)
                        + bp_ref[...])

    return pl.pallas_call(
        body,
        out_shape=jax.ShapeDtypeStruct((n, c), jnp.float32),
    )(x, wp, bp)


def kernel(x, edge_index, Wq, Wk, Wv, Wo, gamma_attn, beta_attn, gamma1,
           beta1, W1, b1, W2, b2, gamma2, beta2, Wp, bp):
    n, d = x.shape
    nl = Wq.shape[0]
    h = 8
    dh = d // h
    accw = d + 16
    scale = 1.0 / (dh ** 0.5)
    src = edge_index[0]
    dst = edge_index[1]
    npad = ((n + 127) // 128) * 128
    zeros = jnp.zeros((npad // _NS, accw), jnp.float32)
    for l in range(nl):
        q, k, v = _tc_qkv(x, Wq[l], Wk[l], Wv[l], scale)
        acc2 = _sc_edge_pass(q, k, v, src, dst, zeros, n, d, h, accw)
        x = _tc_dense(x, acc2, Wo[l], gamma_attn[l], beta_attn[l],
                      gamma1[l], beta1[l], W1[l], b1[l], W2[l], b2[l],
                      gamma2[l], beta2[l])
    return _tc_final(x, Wp, bp)
